# baseline (speedup 1.0000x reference)
import functools

import jax
import jax.numpy as jnp
from jax.experimental import pallas as pl
from jax.experimental.pallas import tpu as pltpu

_BPG = 2


def _ggnn_kernel(h_ref, edge_ref, adj_ref,
                 wh5_ref, whm3_ref, we4_ref, wread_ref, bias_ref,
                 out_ref, ew_ref, *, num_layers, n_nodes, fdim):
    f32 = jnp.float32
    F = fdim
    N = n_nodes

    E = edge_ref.shape[-1]
    adjc = [adj_ref[p].astype(f32) for p in range(_BPG)]
    e2s = [edge_ref[p].reshape(N * N, E) for p in range(_BPG)]
    zrow = jnp.zeros((1, N), f32)
    zpad = jnp.zeros((6, 2 * N), f32)
    for t in range(N // 2):
        for p in range(_BPG):
            adj = adjc[p]
            top = jnp.concatenate([adj[2 * t:2 * t + 1, :], zrow], axis=1)
            bot = jnp.concatenate([zrow, adj[2 * t + 1:2 * t + 2, :]], axis=1)
            lhs = jnp.concatenate([top, bot, zpad], axis=0)
            part = jnp.dot(lhs, e2s[p][2 * N * t:2 * N * (t + 1), :],
                           preferred_element_type=f32)
            ew_ref[p, 2 * t:2 * t + 2, :] = part[0:2, :]

    L = num_layers
    mb_ref = lambda l: bias_ref[l:l + 1, :F]
    brz_ref = lambda l: bias_ref[L + l:L + l + 1, :2 * F]
    bin_ref = lambda l: bias_ref[2 * L + l:2 * L + l + 1, :F]
    bhn_ref = lambda l: bias_ref[3 * L + l:3 * L + l + 1, :F]
    bl1_ref = bias_ref[4 * L:4 * L + 1, :F]
    bl2_ref = bias_ref[4 * L + 1:4 * L + 2, :F]
    inv_n = 1.0 / float(N)

    h0s = [h_ref[p].astype(f32) for p in range(_BPG)]
    adjs = [adj_ref[p].astype(f32) for p in range(_BPG)]
    degs = [jnp.sum(a, axis=1, keepdims=True) for a in adjs]
    ecs = [jnp.dot(ew_ref[p], we4_ref[...], preferred_element_type=f32)
           for p in range(_BPG)]

    hs = list(h0s)
    for l in range(num_layers):
        for p in range(_BPG):
            ph = jnp.dot(hs[p], wh5_ref[l], preferred_element_type=f32)
            agg = jnp.dot(adjs[p], ph[:, :F], preferred_element_type=f32)
            m = (agg + ecs[p][:, l * F:(l + 1) * F]
                 + degs[p] * (ph[:, F:2 * F] + mb_ref(l))) * inv_n

            pm = jnp.dot(m, whm3_ref[l], preferred_element_type=f32)

            rz = jax.nn.sigmoid(ph[:, 2 * F:4 * F] + pm[:, :2 * F] + brz_ref(l))
            r = rz[:, :F]
            z = rz[:, F:]
            n = jnp.tanh(ph[:, 4 * F:] + bin_ref(l)
                         + r * (pm[:, 2 * F:] + bhn_ref(l)))
            hs[p] = jnp.maximum((1.0 - z) * n + z * m, 0.0)

    for p in range(_BPG):
        gl = jnp.dot(jnp.concatenate([hs[p], h0s[p]], axis=1), wread_ref[...],
                     preferred_element_type=f32)
        g = jax.nn.sigmoid(gl[:, :F] + bl1_ref)
        hl2 = gl[:, F:] + bl2_ref
        r_out = jnp.mean(g * hl2, axis=0, keepdims=True)
        out_ref[p, :, :] = jnp.maximum(r_out, 0.0).astype(out_ref.dtype)


def _pack(layers, L1, bL1, L2, bL2, fdim, edim):
    F, E = fdim, edim
    wh5 = jnp.stack([
        jnp.concatenate([lp["W"][:, :F].T, lp["W"][:, F + E:].T,
                         lp["Wih"][0:F].T, lp["Wih"][F:2 * F].T,
                         lp["Wih"][2 * F:].T], axis=1)
        for lp in layers])
    whm3 = jnp.stack([
        jnp.concatenate([lp["Whh"][0:F].T, lp["Whh"][F:2 * F].T,
                         lp["Whh"][2 * F:].T], axis=1)
        for lp in layers])
    we4 = jnp.concatenate([lp["W"][:, F:F + E].T for lp in layers], axis=1)
    wread = jnp.concatenate([
        jnp.concatenate([L1[:, :F].T, L2.T], axis=1),
        jnp.concatenate([L1[:, F:].T, jnp.zeros((F, F), jnp.float32)], axis=1),
    ], axis=0)
    def wide(x):
        x = x.reshape(1, -1)
        return jnp.pad(x, ((0, 0), (0, 2 * F - x.shape[1])))
    bias = jnp.concatenate(
        [wide(lp["Wb"]) for lp in layers]
        + [wide(lp["bih"][:2 * F] + lp["bhh"][:2 * F]) for lp in layers]
        + [wide(lp["bih"][2 * F:]) for lp in layers]
        + [wide(lp["bhh"][2 * F:]) for lp in layers]
        + [wide(bL1), wide(bL2)] + [wide(bL2)] * 6, axis=0)
    return (wh5, whm3, we4, wread, bias)


def kernel(h, edge, adj,
           ly0_W, ly0_Wb, ly0_Wih, ly0_Whh, ly0_bih, ly0_bhh,
           ly1_W, ly1_Wb, ly1_Wih, ly1_Whh, ly1_bih, ly1_bhh,
           ly2_W, ly2_Wb, ly2_Wih, ly2_Whh, ly2_bih, ly2_bhh,
           ly3_W, ly3_Wb, ly3_Wih, ly3_Whh, ly3_bih, ly3_bhh,
           L1, bL1, L2, bL2):
    B, N, F = h.shape
    E = edge.shape[-1]
    layers = [
        {"W": ly0_W, "Wb": ly0_Wb, "Wih": ly0_Wih, "Whh": ly0_Whh,
         "bih": ly0_bih, "bhh": ly0_bhh},
        {"W": ly1_W, "Wb": ly1_Wb, "Wih": ly1_Wih, "Whh": ly1_Whh,
         "bih": ly1_bih, "bhh": ly1_bhh},
        {"W": ly2_W, "Wb": ly2_Wb, "Wih": ly2_Wih, "Whh": ly2_Whh,
         "bih": ly2_bih, "bhh": ly2_bhh},
        {"W": ly3_W, "Wb": ly3_Wb, "Wih": ly3_Wih, "Whh": ly3_Whh,
         "bih": ly3_bih, "bhh": ly3_bhh},
    ]
    L = len(layers)
    packed = _pack(layers, L1, bL1, L2, bL2, F, E)

    body = functools.partial(_ggnn_kernel, num_layers=L, n_nodes=N, fdim=F)

    flops_per_b = (L * (2 * N * F * 5 * F + 2 * N * N * F + 2 * N * F * 3 * F
                        + 20 * N * F)
                   + 2 * N * N * E + 2 * N * E * L * F + 2 * N * 2 * F * 2 * F
                   + 10 * N * F)
    transc_per_b = L * 3 * N * F + N * F
    in_bytes = sum(int(x.size) * x.dtype.itemsize
                   for x in (h, edge, adj) + packed)
    cost = pl.CostEstimate(flops=int(B * flops_per_b),
                           transcendentals=int(B * transc_per_b),
                           bytes_accessed=int(in_bytes + B * F * 4))

    G = _BPG
    out = pl.pallas_call(
        body,
        out_shape=jax.ShapeDtypeStruct((B, 1, F), h.dtype),
        grid_spec=pltpu.PrefetchScalarGridSpec(
            num_scalar_prefetch=0,
            grid=(B // G,),
            in_specs=[
                pl.BlockSpec((G, N, F), lambda b: (b, 0, 0)),
                pl.BlockSpec((G, N, N, E), lambda b: (b, 0, 0, 0)),
                pl.BlockSpec((G, N, N), lambda b: (b, 0, 0)),
                pl.BlockSpec((L, F, 5 * F), lambda b: (0, 0, 0)),
                pl.BlockSpec((L, F, 3 * F), lambda b: (0, 0, 0)),
                pl.BlockSpec((E, L * F), lambda b: (0, 0)),
                pl.BlockSpec((2 * F, 2 * F), lambda b: (0, 0)),
                pl.BlockSpec((24, 2 * F), lambda b: (0, 0)),
            ],
            out_specs=pl.BlockSpec((G, 1, F), lambda b: (b, 0, 0)),
            scratch_shapes=[pltpu.VMEM((G, N, E), jnp.float32)],
        ),
        compiler_params=pltpu.CompilerParams(
            dimension_semantics=("parallel",),
        ),
        cost_estimate=cost,
    )(h, edge, adj, *packed)
    return out.reshape(B, F)


# speedup vs baseline: 1.0019x; 1.0019x over previous
import functools

import jax
import jax.numpy as jnp
from jax.experimental import pallas as pl
from jax.experimental.pallas import tpu as pltpu

_BPG = 2


def _ggnn_kernel(h_ref, edge_ref, adj_ref,
                 wh5_ref, whm3_ref, we4_ref, wread_ref, bias_ref,
                 out_ref, ew_ref, *, num_layers, n_nodes, fdim):
    f32 = jnp.float32
    F = fdim
    N = n_nodes

    for p in range(_BPG):
        adj = adj_ref[p].astype(f32)

        E = edge_ref.shape[-1]
        e2 = edge_ref[p].reshape(N * N, E)
        zrow = jnp.zeros((1, N), f32)
        zpad = jnp.zeros((6, 2 * N), f32)
        for t in range(N // 2):
            top = jnp.concatenate([adj[2 * t:2 * t + 1, :], zrow], axis=1)
            bot = jnp.concatenate([zrow, adj[2 * t + 1:2 * t + 2, :]], axis=1)
            lhs = jnp.concatenate([top, bot, zpad], axis=0)
            part = jnp.dot(lhs, e2[2 * N * t:2 * N * (t + 1), :],
                           preferred_element_type=f32)
            ew_ref[p, 2 * t:2 * t + 2, :] = part[0:2, :]

    L = num_layers
    mb_ref = lambda l: bias_ref[l:l + 1, :F]
    brz_ref = lambda l: bias_ref[L + l:L + l + 1, :2 * F]
    bin_ref = lambda l: bias_ref[2 * L + l:2 * L + l + 1, :F]
    bhn_ref = lambda l: bias_ref[3 * L + l:3 * L + l + 1, :F]
    bl1_ref = bias_ref[4 * L:4 * L + 1, :F]
    bl2_ref = bias_ref[4 * L + 1:4 * L + 2, :F]
    inv_n = 1.0 / float(N)

    h0s = [h_ref[p].astype(f32) for p in range(_BPG)]
    adjs = [adj_ref[p].astype(f32) for p in range(_BPG)]
    degs = [jnp.sum(a, axis=1, keepdims=True) for a in adjs]
    ecs = [jnp.dot(ew_ref[p], we4_ref[...], preferred_element_type=f32)
           for p in range(_BPG)]

    hs = list(h0s)
    for l in range(num_layers):
        for p in range(_BPG):
            ph = jnp.dot(hs[p], wh5_ref[l], preferred_element_type=f32)
            agg = jnp.dot(adjs[p], ph[:, :F], preferred_element_type=f32)
            m = (agg + ecs[p][:, l * F:(l + 1) * F]
                 + degs[p] * (ph[:, F:2 * F] + mb_ref(l))) * inv_n

            pm = jnp.dot(m, whm3_ref[l], preferred_element_type=f32)

            rz = jax.nn.sigmoid(ph[:, 2 * F:4 * F] + pm[:, :2 * F] + brz_ref(l))
            r = rz[:, :F]
            z = rz[:, F:]
            n = jnp.tanh(ph[:, 4 * F:] + bin_ref(l)
                         + r * (pm[:, 2 * F:] + bhn_ref(l)))
            hs[p] = jnp.maximum((1.0 - z) * n + z * m, 0.0)

    for p in range(_BPG):
        gl = jnp.dot(jnp.concatenate([hs[p], h0s[p]], axis=1), wread_ref[...],
                     preferred_element_type=f32)
        g = jax.nn.sigmoid(gl[:, :F] + bl1_ref)
        hl2 = gl[:, F:] + bl2_ref
        r_out = jnp.mean(g * hl2, axis=0, keepdims=True)
        out_ref[p, :, :] = jnp.maximum(r_out, 0.0).astype(out_ref.dtype)


def _pack(layers, L1, bL1, L2, bL2, fdim, edim):
    F, E = fdim, edim
    wh5 = jnp.stack([
        jnp.concatenate([lp["W"][:, :F].T, lp["W"][:, F + E:].T,
                         lp["Wih"][0:F].T, lp["Wih"][F:2 * F].T,
                         lp["Wih"][2 * F:].T], axis=1)
        for lp in layers])
    whm3 = jnp.stack([
        jnp.concatenate([lp["Whh"][0:F].T, lp["Whh"][F:2 * F].T,
                         lp["Whh"][2 * F:].T], axis=1)
        for lp in layers])
    we4 = jnp.concatenate([lp["W"][:, F:F + E].T for lp in layers], axis=1)
    wread = jnp.concatenate([
        jnp.concatenate([L1[:, :F].T, L2.T], axis=1),
        jnp.concatenate([L1[:, F:].T, jnp.zeros((F, F), jnp.float32)], axis=1),
    ], axis=0)
    def wide(x):
        x = x.reshape(1, -1)
        return jnp.pad(x, ((0, 0), (0, 2 * F - x.shape[1])))
    bias = jnp.concatenate(
        [wide(lp["Wb"]) for lp in layers]
        + [wide(lp["bih"][:2 * F] + lp["bhh"][:2 * F]) for lp in layers]
        + [wide(lp["bih"][2 * F:]) for lp in layers]
        + [wide(lp["bhh"][2 * F:]) for lp in layers]
        + [wide(bL1), wide(bL2)] + [wide(bL2)] * 6, axis=0)
    return (wh5, whm3, we4, wread, bias)


def kernel(h, edge, adj,
           ly0_W, ly0_Wb, ly0_Wih, ly0_Whh, ly0_bih, ly0_bhh,
           ly1_W, ly1_Wb, ly1_Wih, ly1_Whh, ly1_bih, ly1_bhh,
           ly2_W, ly2_Wb, ly2_Wih, ly2_Whh, ly2_bih, ly2_bhh,
           ly3_W, ly3_Wb, ly3_Wih, ly3_Whh, ly3_bih, ly3_bhh,
           L1, bL1, L2, bL2):
    B, N, F = h.shape
    E = edge.shape[-1]
    layers = [
        {"W": ly0_W, "Wb": ly0_Wb, "Wih": ly0_Wih, "Whh": ly0_Whh,
         "bih": ly0_bih, "bhh": ly0_bhh},
        {"W": ly1_W, "Wb": ly1_Wb, "Wih": ly1_Wih, "Whh": ly1_Whh,
         "bih": ly1_bih, "bhh": ly1_bhh},
        {"W": ly2_W, "Wb": ly2_Wb, "Wih": ly2_Wih, "Whh": ly2_Whh,
         "bih": ly2_bih, "bhh": ly2_bhh},
        {"W": ly3_W, "Wb": ly3_Wb, "Wih": ly3_Wih, "Whh": ly3_Whh,
         "bih": ly3_bih, "bhh": ly3_bhh},
    ]
    L = len(layers)
    packed = _pack(layers, L1, bL1, L2, bL2, F, E)

    body = functools.partial(_ggnn_kernel, num_layers=L, n_nodes=N, fdim=F)

    flops_per_b = (L * (2 * N * F * 5 * F + 2 * N * N * F + 2 * N * F * 3 * F
                        + 20 * N * F)
                   + 2 * N * N * E + 2 * N * E * L * F + 2 * N * 2 * F * 2 * F
                   + 10 * N * F)
    transc_per_b = L * 3 * N * F + N * F
    in_bytes = sum(int(x.size) * x.dtype.itemsize
                   for x in (h, edge, adj) + packed)
    cost = pl.CostEstimate(flops=int(B * flops_per_b),
                           transcendentals=int(B * transc_per_b),
                           bytes_accessed=int(in_bytes + B * F * 4))

    G = _BPG
    out = pl.pallas_call(
        body,
        out_shape=jax.ShapeDtypeStruct((B, 1, F), h.dtype),
        grid_spec=pltpu.PrefetchScalarGridSpec(
            num_scalar_prefetch=0,
            grid=(B // G,),
            in_specs=[
                pl.BlockSpec((G, N, F), lambda b: (b, 0, 0)),
                pl.BlockSpec((G, N, N, E), lambda b: (b, 0, 0, 0)),
                pl.BlockSpec((G, N, N), lambda b: (b, 0, 0)),
                pl.BlockSpec((L, F, 5 * F), lambda b: (0, 0, 0)),
                pl.BlockSpec((L, F, 3 * F), lambda b: (0, 0, 0)),
                pl.BlockSpec((E, L * F), lambda b: (0, 0)),
                pl.BlockSpec((2 * F, 2 * F), lambda b: (0, 0)),
                pl.BlockSpec((24, 2 * F), lambda b: (0, 0)),
            ],
            out_specs=pl.BlockSpec((G, 1, F), lambda b: (b, 0, 0)),
            scratch_shapes=[pltpu.VMEM((G, N, E), jnp.float32)],
        ),
        compiler_params=pltpu.CompilerParams(
            dimension_semantics=("parallel",),
        ),
        cost_estimate=cost,
    )(h, edge, adj, *packed)
    return out.reshape(B, F)
